# baseline (speedup 1.0000x reference)
import functools

import jax
import jax.numpy as jnp
from jax import lax
from jax.experimental import pallas as pl
from jax.experimental.pallas import tpu as pltpu
from jax.experimental.pallas import tpu_sc as plsc

NC, NS = 2, 16
NW = NC * NS
CH = 64
NBUF = 4
LOOK = 2
NIDX = 8
IDXLOOK = 6
DUMMY = 8


def _sc_aggregate(ed, x, zeros):
    n, d = x.shape
    nchunk = ed.shape[1]
    rpt = (n // NS) // 8 * 8
    tail = n - NS * rpt

    mesh = plsc.VectorSubcoreMesh(core_axis_name="c", subcore_axis_name="s")

    @functools.partial(
        pl.kernel,
        mesh=mesh,
        out_type=jax.ShapeDtypeStruct((NC, n, d), jnp.float32),
        scratch_types=(
            [pltpu.VMEM((NIDX, 2, CH), jnp.int32)]
            + [pltpu.VMEM((CH, d), jnp.float32) for _ in range(NBUF)]
            + [pltpu.VMEM_SHARED((n + DUMMY, d), jnp.float32)]
            + [pltpu.SemaphoreType.DMA for _ in range(NIDX + 2 * NBUF)]
        ),
    )
    def agg_kernel(ed_hbm, x_hbm, zeros_hbm, out_hbm, *scr):
        idx_v = scr[0]
        rows = scr[1:1 + NBUF]
        agg_sh = scr[1 + NBUF]
        isem = scr[2 + NBUF:2 + NBUF + NIDX]
        gsem = scr[2 + NBUF + NIDX:2 + 2 * NBUF + NIDX]
        ssem = scr[2 + 2 * NBUF + NIDX:2 + 3 * NBUF + NIDX]

        c = lax.axis_index("c")
        s = lax.axis_index("s")
        wid = c * NS + s

        pltpu.sync_copy(zeros_hbm, agg_sh.at[pl.ds(s * rpt, rpt)])
        if tail:
            @pl.when(s == 0)
            def _():
                pltpu.sync_copy(zeros_hbm.at[pl.ds(0, tail)],
                                agg_sh.at[pl.ds(NS * rpt, tail)])
        plsc.subcore_barrier()

        def fire_idx(i, sl):
            pltpu.async_copy(ed_hbm.at[wid, i], idx_v.at[sl], isem[sl])

        def wait_idx(sl):
            pltpu.make_async_copy(ed_hbm.at[wid, 0], idx_v.at[sl],
                                  isem[sl]).wait()

        def fire_gather(sl, j):
            pltpu.async_copy(x_hbm.at[idx_v.at[sl, 0]], rows[j], gsem[j])

        def wait_gather(j):
            pltpu.make_async_copy(x_hbm.at[idx_v.at[0, 0]],
                                  rows[j], gsem[j]).wait()

        def fire_scatter(sl, j):
            pltpu.async_copy(rows[j], agg_sh.at[idx_v.at[sl, 1]],
                             ssem[j], add=True)

        def wait_scatter(sl, j):
            pltpu.make_async_copy(rows[j], agg_sh.at[idx_v.at[sl, 1]],
                                  ssem[j]).wait()

        def step(i, ii, edge=False):
            wait_gather(ii % NBUF)
            if not edge or ii - 1 >= 0:
                wait_scatter((ii - 1) % NIDX, (ii - 1) % NBUF)
            fire_scatter(ii % NIDX, ii % NBUF)
            if not edge or ii + IDXLOOK < nchunk:
                fire_idx(i + IDXLOOK, (ii + IDXLOOK) % NIDX)
            if not edge or ii + LOOK < nchunk:
                wait_idx((ii + LOOK) % NIDX)
                fire_gather((ii + LOOK) % NIDX, (ii + LOOK) % NBUF)

        for f in range(IDXLOOK):
            fire_idx(f, f)
        for i in range(LOOK):
            wait_idx(i)
            fire_gather(i, i)

        n_main = (nchunk - IDXLOOK - 2 * LOOK) // NIDX
        head = nchunk - IDXLOOK - NIDX * n_main
        for i in range(head):
            step(i, i, edge=True)

        def body(k, carry):
            base = NIDX * k + head
            for m in range(NIDX):
                step(base + m, head + m)
            return carry

        lax.fori_loop(0, n_main, body, 0)

        for i in range(nchunk - IDXLOOK, nchunk):
            step(i, i, edge=True)
        wait_scatter((nchunk - 1) % NIDX, (nchunk - 1) % NBUF)

        plsc.subcore_barrier()

        pltpu.sync_copy(agg_sh.at[pl.ds(s * rpt, rpt)],
                        out_hbm.at[c, pl.ds(s * rpt, rpt)])
        if tail:
            @pl.when(s == 0)
            def _():
                pltpu.sync_copy(agg_sh.at[pl.ds(NS * rpt, tail)],
                                out_hbm.at[c, pl.ds(NS * rpt, tail)])

    return agg_kernel(ed, x, zeros)


def _tc_block(x_ref, a0_ref, a1_ref, w1_ref, b1_ref, w2_ref, b2_ref,
              g_ref, be_ref, o_ref):
    xb = x_ref[...]
    h = xb + a0_ref[0] + a1_ref[0]
    t = jnp.dot(h, w1_ref[...], preferred_element_type=jnp.float32) + b1_ref[...]
    t = jnp.maximum(t, 0.0)
    t = jnp.dot(t, w2_ref[...], preferred_element_type=jnp.float32) + b2_ref[...]
    mean = jnp.mean(t, axis=-1, keepdims=True)
    cent = t - mean
    var = jnp.mean(cent * cent, axis=-1, keepdims=True)
    t = cent * lax.rsqrt(var + 1e-5) * g_ref[...] + be_ref[...]
    o_ref[...] = xb + jnp.maximum(t, 0.0)


def _tc_mlp(x, agg, W1, b1, W2, b2, gamma, beta, block_rows=400):
    n, d = x.shape
    grid = (n // block_rows,)
    row_spec = pl.BlockSpec((block_rows, d), lambda i: (i, 0))
    a0_spec = pl.BlockSpec((1, block_rows, d), lambda i: (0, i, 0))
    a1_spec = pl.BlockSpec((1, block_rows, d), lambda i: (1, i, 0))
    full_spec = pl.BlockSpec((d, d), lambda i: (0, 0))
    vec_spec = pl.BlockSpec((1, d), lambda i: (0, 0))
    return pl.pallas_call(
        _tc_block,
        grid=grid,
        in_specs=[row_spec, a0_spec, a1_spec, full_spec, vec_spec,
                  full_spec, vec_spec, vec_spec, vec_spec],
        out_specs=row_spec,
        out_shape=jax.ShapeDtypeStruct((n, d), jnp.float32),
    )(x, agg, agg, W1, b1.reshape(1, d), W2, b2.reshape(1, d),
      gamma.reshape(1, d), beta.reshape(1, d))


def kernel(x, edge_index, W1, b1, W2, b2, gamma, beta):
    n, d = x.shape
    e = edge_index.shape[1]
    quantum = NW * CH * NIDX
    e_pad = -(-e // quantum) * quantum
    pad = e_pad - e
    src_flat = edge_index[0]
    dst_flat = edge_index[1]
    if pad:
        src_flat = jnp.concatenate(
            [src_flat, jnp.zeros((pad,), jnp.int32)])
        dst_flat = jnp.concatenate(
            [dst_flat, n + (jnp.arange(pad, dtype=jnp.int32) % DUMMY)])
    epw = e_pad // NW
    nchunk = epw // CH
    ed = jnp.stack([src_flat.reshape(NW, nchunk, CH),
                    dst_flat.reshape(NW, nchunk, CH)], axis=2)
    zeros = jnp.zeros(((n // NS) // 8 * 8, d), dtype=jnp.float32)
    agg = _sc_aggregate(ed, x, zeros)
    return _tc_mlp(x, agg, W1, b1, W2, b2, gamma, beta)


# speedup vs baseline: 1.0017x; 1.0017x over previous
import functools

import jax
import jax.numpy as jnp
from jax import lax
from jax.experimental import pallas as pl
from jax.experimental.pallas import tpu as pltpu
from jax.experimental.pallas import tpu_sc as plsc

NC, NS = 2, 16
NW = NC * NS
CH = 64
NBUF = 4
LOOK = 2
NIDX = 8
IDXLOOK = 6
DUMMY = 64


def _sc_aggregate(ed, x, zeros):
    n, d = x.shape
    nchunk = ed.shape[1]
    rpt = (n // NS) // 8 * 8
    tail = n - NS * rpt

    mesh = plsc.VectorSubcoreMesh(core_axis_name="c", subcore_axis_name="s")

    @functools.partial(
        pl.kernel,
        mesh=mesh,
        out_type=jax.ShapeDtypeStruct((NC, n, d), jnp.float32),
        scratch_types=(
            [pltpu.VMEM((NIDX, 2, CH), jnp.int32)]
            + [pltpu.VMEM((CH, d), jnp.float32) for _ in range(NBUF)]
            + [pltpu.VMEM_SHARED((n + DUMMY, d), jnp.float32)]
            + [pltpu.SemaphoreType.DMA for _ in range(NIDX + 2 * NBUF)]
        ),
    )
    def agg_kernel(ed_hbm, x_hbm, zeros_hbm, out_hbm, *scr):
        idx_v = scr[0]
        rows = scr[1:1 + NBUF]
        agg_sh = scr[1 + NBUF]
        isem = scr[2 + NBUF:2 + NBUF + NIDX]
        gsem = scr[2 + NBUF + NIDX:2 + 2 * NBUF + NIDX]
        ssem = scr[2 + 2 * NBUF + NIDX:2 + 3 * NBUF + NIDX]

        c = lax.axis_index("c")
        s = lax.axis_index("s")
        wid = c * NS + s

        pltpu.sync_copy(zeros_hbm, agg_sh.at[pl.ds(s * rpt, rpt)])
        if tail:
            @pl.when(s == 0)
            def _():
                pltpu.sync_copy(zeros_hbm.at[pl.ds(0, tail)],
                                agg_sh.at[pl.ds(NS * rpt, tail)])
        plsc.subcore_barrier()

        def fire_idx(i, sl):
            pltpu.async_copy(ed_hbm.at[wid, i], idx_v.at[sl], isem[sl])

        def wait_idx(sl):
            pltpu.make_async_copy(ed_hbm.at[wid, 0], idx_v.at[sl],
                                  isem[sl]).wait()

        def fire_gather(sl, j):
            pltpu.async_copy(x_hbm.at[idx_v.at[sl, 0]], rows[j], gsem[j])

        def wait_gather(j):
            pltpu.make_async_copy(x_hbm.at[idx_v.at[0, 0]],
                                  rows[j], gsem[j]).wait()

        def fire_scatter(sl, j):
            pltpu.async_copy(rows[j], agg_sh.at[idx_v.at[sl, 1]],
                             ssem[j], add=True)

        def wait_scatter(sl, j):
            pltpu.make_async_copy(rows[j], agg_sh.at[idx_v.at[sl, 1]],
                                  ssem[j]).wait()

        def step(i, ii, edge=False):
            wait_gather(ii % NBUF)
            if not edge or ii - 1 >= 0:
                wait_scatter((ii - 1) % NIDX, (ii - 1) % NBUF)
            fire_scatter(ii % NIDX, ii % NBUF)
            if not edge or ii + IDXLOOK < nchunk:
                fire_idx(i + IDXLOOK, (ii + IDXLOOK) % NIDX)
            if not edge or ii + LOOK < nchunk:
                wait_idx((ii + LOOK) % NIDX)
                fire_gather((ii + LOOK) % NIDX, (ii + LOOK) % NBUF)

        for f in range(IDXLOOK):
            fire_idx(f, f)
        for i in range(LOOK):
            wait_idx(i)
            fire_gather(i, i)

        n_main = (nchunk - IDXLOOK - 2 * LOOK) // NIDX
        head = nchunk - IDXLOOK - NIDX * n_main
        for i in range(head):
            step(i, i, edge=True)

        def body(k, carry):
            base = NIDX * k + head
            for m in range(NIDX):
                step(base + m, head + m)
            return carry

        lax.fori_loop(0, n_main, body, 0)

        for i in range(nchunk - IDXLOOK, nchunk):
            step(i, i, edge=True)
        wait_scatter((nchunk - 1) % NIDX, (nchunk - 1) % NBUF)

        plsc.subcore_barrier()

        pltpu.sync_copy(agg_sh.at[pl.ds(s * rpt, rpt)],
                        out_hbm.at[c, pl.ds(s * rpt, rpt)])
        if tail:
            @pl.when(s == 0)
            def _():
                pltpu.sync_copy(agg_sh.at[pl.ds(NS * rpt, tail)],
                                out_hbm.at[c, pl.ds(NS * rpt, tail)])

    return agg_kernel(ed, x, zeros)


def _tc_block(x_ref, a0_ref, a1_ref, w1_ref, b1_ref, w2_ref, b2_ref,
              g_ref, be_ref, o_ref):
    xb = x_ref[...]
    h = xb + a0_ref[0] + a1_ref[0]
    t = jnp.dot(h, w1_ref[...], preferred_element_type=jnp.float32) + b1_ref[...]
    t = jnp.maximum(t, 0.0)
    t = jnp.dot(t, w2_ref[...], preferred_element_type=jnp.float32) + b2_ref[...]
    mean = jnp.mean(t, axis=-1, keepdims=True)
    cent = t - mean
    var = jnp.mean(cent * cent, axis=-1, keepdims=True)
    t = cent * lax.rsqrt(var + 1e-5) * g_ref[...] + be_ref[...]
    o_ref[...] = xb + jnp.maximum(t, 0.0)


def _tc_mlp(x, agg, W1, b1, W2, b2, gamma, beta, block_rows=400):
    n, d = x.shape
    grid = (n // block_rows,)
    row_spec = pl.BlockSpec((block_rows, d), lambda i: (i, 0))
    a0_spec = pl.BlockSpec((1, block_rows, d), lambda i: (0, i, 0))
    a1_spec = pl.BlockSpec((1, block_rows, d), lambda i: (1, i, 0))
    full_spec = pl.BlockSpec((d, d), lambda i: (0, 0))
    vec_spec = pl.BlockSpec((1, d), lambda i: (0, 0))
    return pl.pallas_call(
        _tc_block,
        grid=grid,
        in_specs=[row_spec, a0_spec, a1_spec, full_spec, vec_spec,
                  full_spec, vec_spec, vec_spec, vec_spec],
        out_specs=row_spec,
        out_shape=jax.ShapeDtypeStruct((n, d), jnp.float32),
    )(x, agg, agg, W1, b1.reshape(1, d), W2, b2.reshape(1, d),
      gamma.reshape(1, d), beta.reshape(1, d))


def kernel(x, edge_index, W1, b1, W2, b2, gamma, beta):
    n, d = x.shape
    e = edge_index.shape[1]
    quantum = NW * CH * NIDX
    e_pad = -(-e // quantum) * quantum
    pad = e_pad - e
    src_flat = edge_index[0]
    dst_flat = edge_index[1]
    if pad:
        src_flat = jnp.concatenate(
            [src_flat, jnp.zeros((pad,), jnp.int32)])
        dst_flat = jnp.concatenate(
            [dst_flat, n + (jnp.arange(pad, dtype=jnp.int32) % DUMMY)])
    epw = e_pad // NW
    nchunk = epw // CH
    ed = jnp.stack([src_flat.reshape(NW, nchunk, CH),
                    dst_flat.reshape(NW, nchunk, CH)], axis=2)
    zeros = jnp.zeros(((n // NS) // 8 * 8, d), dtype=jnp.float32)
    agg = _sc_aggregate(ed, x, zeros)
    return _tc_mlp(x, agg, W1, b1, W2, b2, gamma, beta)


# speedup vs baseline: 1.0156x; 1.0138x over previous
import functools

import jax
import jax.numpy as jnp
from jax import lax
from jax.experimental import pallas as pl
from jax.experimental.pallas import tpu as pltpu
from jax.experimental.pallas import tpu_sc as plsc

NC, NS = 2, 16
NW = NC * NS
CH = 64
NBUF = 4
LOOK = 2
NIDX = 8
IDXLOOK = 6
DUMMY = 64


def _sc_aggregate(src3, dst3, x, zeros):
    n, d = x.shape
    nchunk = src3.shape[1]
    rpt = (n // NS) // 8 * 8
    tail = n - NS * rpt

    mesh = plsc.VectorSubcoreMesh(core_axis_name="c", subcore_axis_name="s")

    @functools.partial(
        pl.kernel,
        mesh=mesh,
        out_type=jax.ShapeDtypeStruct((NC, n, d), jnp.float32),
        scratch_types=(
            [pltpu.VMEM((NIDX, CH), jnp.int32),
             pltpu.VMEM((NIDX, CH), jnp.int32)]
            + [pltpu.VMEM((CH, d), jnp.float32) for _ in range(NBUF)]
            + [pltpu.VMEM_SHARED((n + DUMMY, d), jnp.float32)]
            + [pltpu.SemaphoreType.DMA for _ in range(2 * NIDX + 2 * NBUF)]
        ),
    )
    def agg_kernel(src_hbm, dst_hbm, x_hbm, zeros_hbm, out_hbm, *scr):
        src_ring, dst_ring = scr[0], scr[1]
        rows = scr[2:2 + NBUF]
        agg_sh = scr[2 + NBUF]
        base = 3 + NBUF
        isem_s = scr[base:base + NIDX]
        isem_d = scr[base + NIDX:base + 2 * NIDX]
        gsem = scr[base + 2 * NIDX:base + 2 * NIDX + NBUF]
        ssem = scr[base + 2 * NIDX + NBUF:base + 2 * NIDX + 2 * NBUF]

        c = lax.axis_index("c")
        s = lax.axis_index("s")
        wid = c * NS + s

        pltpu.sync_copy(zeros_hbm, agg_sh.at[pl.ds(s * rpt, rpt)])
        if tail:
            @pl.when(s == 0)
            def _():
                pltpu.sync_copy(zeros_hbm.at[pl.ds(0, tail)],
                                agg_sh.at[pl.ds(NS * rpt, tail)])
        plsc.subcore_barrier()

        def fire_idx(i, sl):
            pltpu.async_copy(src_hbm.at[wid, i], src_ring.at[sl], isem_s[sl])
            pltpu.async_copy(dst_hbm.at[wid, i], dst_ring.at[sl], isem_d[sl])

        def wait_idx(sl):
            pltpu.make_async_copy(src_hbm.at[wid, 0], src_ring.at[sl],
                                  isem_s[sl]).wait()
            pltpu.make_async_copy(dst_hbm.at[wid, 0], dst_ring.at[sl],
                                  isem_d[sl]).wait()

        def fire_gather(sl, j):
            pltpu.async_copy(x_hbm.at[src_ring.at[sl]], rows[j], gsem[j])

        def wait_gather(j):
            pltpu.make_async_copy(x_hbm.at[src_ring.at[0]],
                                  rows[j], gsem[j]).wait()

        def fire_scatter(sl, j):
            pltpu.async_copy(rows[j], agg_sh.at[dst_ring.at[sl]],
                             ssem[j], add=True)

        def wait_scatter(sl, j):
            pltpu.make_async_copy(rows[j], agg_sh.at[dst_ring.at[sl]],
                                  ssem[j]).wait()

        def step(i, ii, edge=False):
            wait_gather(ii % NBUF)
            if not edge or ii - 1 >= 0:
                wait_scatter((ii - 1) % NIDX, (ii - 1) % NBUF)
            fire_scatter(ii % NIDX, ii % NBUF)
            if not edge or ii + IDXLOOK < nchunk:
                fire_idx(i + IDXLOOK, (ii + IDXLOOK) % NIDX)
            if not edge or ii + LOOK < nchunk:
                wait_idx((ii + LOOK) % NIDX)
                fire_gather((ii + LOOK) % NIDX, (ii + LOOK) % NBUF)

        for f in range(IDXLOOK):
            fire_idx(f, f)
        for i in range(LOOK):
            wait_idx(i)
            fire_gather(i, i)

        n_main = (nchunk - IDXLOOK - 2 * LOOK) // NIDX
        head = nchunk - IDXLOOK - NIDX * n_main
        for i in range(head):
            step(i, i, edge=True)

        def body(k, carry):
            base = NIDX * k + head
            for m in range(NIDX):
                step(base + m, head + m)
            return carry

        lax.fori_loop(0, n_main, body, 0)

        for i in range(nchunk - IDXLOOK, nchunk):
            step(i, i, edge=True)
        wait_scatter((nchunk - 1) % NIDX, (nchunk - 1) % NBUF)

        plsc.subcore_barrier()

        pltpu.sync_copy(agg_sh.at[pl.ds(s * rpt, rpt)],
                        out_hbm.at[c, pl.ds(s * rpt, rpt)])
        if tail:
            @pl.when(s == 0)
            def _():
                pltpu.sync_copy(agg_sh.at[pl.ds(NS * rpt, tail)],
                                out_hbm.at[c, pl.ds(NS * rpt, tail)])

    return agg_kernel(src3, dst3, x, zeros)


def _tc_block(x_ref, a0_ref, a1_ref, w1_ref, b1_ref, w2_ref, b2_ref,
              g_ref, be_ref, o_ref):
    xb = x_ref[...]
    h = xb + a0_ref[0] + a1_ref[0]
    t = jnp.dot(h, w1_ref[...], preferred_element_type=jnp.float32) + b1_ref[...]
    t = jnp.maximum(t, 0.0)
    t = jnp.dot(t, w2_ref[...], preferred_element_type=jnp.float32) + b2_ref[...]
    mean = jnp.mean(t, axis=-1, keepdims=True)
    cent = t - mean
    var = jnp.mean(cent * cent, axis=-1, keepdims=True)
    t = cent * lax.rsqrt(var + 1e-5) * g_ref[...] + be_ref[...]
    o_ref[...] = xb + jnp.maximum(t, 0.0)


def _tc_mlp(x, agg, W1, b1, W2, b2, gamma, beta, block_rows=400):
    n, d = x.shape
    grid = (n // block_rows,)
    row_spec = pl.BlockSpec((block_rows, d), lambda i: (i, 0))
    a0_spec = pl.BlockSpec((1, block_rows, d), lambda i: (0, i, 0))
    a1_spec = pl.BlockSpec((1, block_rows, d), lambda i: (1, i, 0))
    full_spec = pl.BlockSpec((d, d), lambda i: (0, 0))
    vec_spec = pl.BlockSpec((1, d), lambda i: (0, 0))
    return pl.pallas_call(
        _tc_block,
        grid=grid,
        in_specs=[row_spec, a0_spec, a1_spec, full_spec, vec_spec,
                  full_spec, vec_spec, vec_spec, vec_spec],
        out_specs=row_spec,
        out_shape=jax.ShapeDtypeStruct((n, d), jnp.float32),
    )(x, agg, agg, W1, b1.reshape(1, d), W2, b2.reshape(1, d),
      gamma.reshape(1, d), beta.reshape(1, d))


def kernel(x, edge_index, W1, b1, W2, b2, gamma, beta):
    n, d = x.shape
    e = edge_index.shape[1]
    quantum = NW * CH * NIDX
    e_pad = -(-e // quantum) * quantum
    pad = e_pad - e
    src_flat = edge_index[0]
    dst_flat = edge_index[1]
    if pad:
        src_flat = jnp.concatenate(
            [src_flat, jnp.zeros((pad,), jnp.int32)])
        dst_flat = jnp.concatenate(
            [dst_flat, n + (jnp.arange(pad, dtype=jnp.int32) % DUMMY)])
    epw = e_pad // NW
    nchunk = epw // CH
    src3 = src_flat.reshape(NW, nchunk, CH)
    dst3 = dst_flat.reshape(NW, nchunk, CH)
    zeros = jnp.zeros(((n // NS) // 8 * 8, d), dtype=jnp.float32)
    agg = _sc_aggregate(src3, dst3, x, zeros)
    return _tc_mlp(x, agg, W1, b1, W2, b2, gamma, beta)


# speedup vs baseline: 3.1783x; 3.1296x over previous
import functools

import jax
import jax.numpy as jnp
from jax import lax
from jax.experimental import pallas as pl
from jax.experimental.pallas import tpu as pltpu
from jax.experimental.pallas import tpu_sc as plsc

NC, NS = 2, 16
NW = NC * NS
CH = 64
NBUF = 4
LOOK = 2
NIDX = 8
IDXLOOK = 6
DUMMY = 64


def _sc_aggregate(src3, dst3, x, zeros):
    n, d = x.shape
    nchunk = src3.shape[1]
    rpt = (n // NS) // 8 * 8
    tail = n - NS * rpt

    mesh = plsc.VectorSubcoreMesh(core_axis_name="c", subcore_axis_name="s")

    @functools.partial(
        pl.kernel,
        mesh=mesh,
        out_type=jax.ShapeDtypeStruct((NC, n, d), jnp.float32),
        scratch_types=(
            [pltpu.VMEM((NIDX, CH), jnp.int32),
             pltpu.VMEM((NIDX, CH), jnp.int32)]
            + [pltpu.VMEM((CH, d), jnp.float32) for _ in range(NBUF)]
            + [pltpu.VMEM_SHARED((n + DUMMY, d), jnp.float32)]
            + [pltpu.SemaphoreType.DMA for _ in range(2 * NIDX + 2 * NBUF)]
        ),
    )
    def agg_kernel(src_hbm, dst_hbm, x_hbm, zeros_hbm, out_hbm, *scr):
        src_ring, dst_ring = scr[0], scr[1]
        rows = scr[2:2 + NBUF]
        agg_sh = scr[2 + NBUF]
        base = 3 + NBUF
        isem_s = scr[base:base + NIDX]
        isem_d = scr[base + NIDX:base + 2 * NIDX]
        gsem = scr[base + 2 * NIDX:base + 2 * NIDX + NBUF]
        ssem = scr[base + 2 * NIDX + NBUF:base + 2 * NIDX + 2 * NBUF]

        c = lax.axis_index("c")
        s = lax.axis_index("s")
        wid = c * NS + s

        pltpu.sync_copy(zeros_hbm, agg_sh.at[pl.ds(s * rpt, rpt)])
        if tail:
            @pl.when(s == 0)
            def _():
                pltpu.sync_copy(zeros_hbm.at[pl.ds(0, tail)],
                                agg_sh.at[pl.ds(NS * rpt, tail)])
        plsc.subcore_barrier()

        def fire_idx(i, sl):
            pltpu.async_copy(src_hbm.at[wid, i], src_ring.at[sl], isem_s[sl])
            pltpu.async_copy(dst_hbm.at[wid, i], dst_ring.at[sl], isem_d[sl])

        def wait_idx(sl):
            pltpu.make_async_copy(src_hbm.at[wid, 0], src_ring.at[sl],
                                  isem_s[sl]).wait()
            pltpu.make_async_copy(dst_hbm.at[wid, 0], dst_ring.at[sl],
                                  isem_d[sl]).wait()

        def fire_gather(sl, j):
            pltpu.async_copy(x_hbm.at[src_ring.at[sl]], rows[j], gsem[j])

        def wait_gather(j):
            pltpu.make_async_copy(x_hbm.at[src_ring.at[0]],
                                  rows[j], gsem[j]).wait()

        def fire_scatter(sl, j):
            pltpu.async_copy(rows[j], agg_sh.at[dst_ring.at[sl]],
                             ssem[j], add=True)

        def wait_scatter(sl, j):
            pltpu.make_async_copy(rows[j], agg_sh.at[dst_ring.at[sl]],
                                  ssem[j]).wait()

        def step(i, ii, edge=False):
            wait_gather(ii % NBUF)
            if not edge or ii - 1 >= 0:
                wait_scatter((ii - 1) % NIDX, (ii - 1) % NBUF)
            fire_scatter(ii % NIDX, ii % NBUF)
            if not edge or ii + IDXLOOK < nchunk:
                fire_idx(i + IDXLOOK, (ii + IDXLOOK) % NIDX)
            if not edge or ii + LOOK < nchunk:
                wait_idx((ii + LOOK) % NIDX)
                fire_gather((ii + LOOK) % NIDX, (ii + LOOK) % NBUF)

        for f in range(IDXLOOK):
            fire_idx(f, f)
        for i in range(LOOK):
            wait_idx(i)
            fire_gather(i, i)

        n_main = (nchunk - IDXLOOK - 2 * LOOK) // NIDX
        head = nchunk - IDXLOOK - NIDX * n_main
        for i in range(head):
            step(i, i, edge=True)

        def body(k, carry):
            base = NIDX * k + head
            for m in range(NIDX):
                step(base + m, head + m)
            return carry

        lax.fori_loop(0, n_main, body, 0)

        for i in range(nchunk - IDXLOOK, nchunk):
            step(i, i, edge=True)
        wait_scatter((nchunk - 1) % NIDX, (nchunk - 1) % NBUF)

        plsc.subcore_barrier()

        pltpu.sync_copy(agg_sh.at[pl.ds(s * rpt, rpt)],
                        out_hbm.at[c, pl.ds(s * rpt, rpt)])
        if tail:
            @pl.when(s == 0)
            def _():
                pltpu.sync_copy(agg_sh.at[pl.ds(NS * rpt, tail)],
                                out_hbm.at[c, pl.ds(NS * rpt, tail)])

    return agg_kernel(src3, dst3, x, zeros)


def _tc_block(x_ref, a0_ref, a1_ref, w1_ref, b1_ref, w2_ref, b2_ref,
              g_ref, be_ref, o_ref):
    xb = x_ref[...]
    h = xb + a0_ref[0] + a1_ref[0]
    t = jnp.dot(h, w1_ref[...], preferred_element_type=jnp.float32) + b1_ref[...]
    t = jnp.maximum(t, 0.0)
    t = jnp.dot(t, w2_ref[...], preferred_element_type=jnp.float32) + b2_ref[...]
    mean = jnp.mean(t, axis=-1, keepdims=True)
    cent = t - mean
    var = jnp.mean(cent * cent, axis=-1, keepdims=True)
    t = cent * lax.rsqrt(var + 1e-5) * g_ref[...] + be_ref[...]
    o_ref[...] = xb + jnp.maximum(t, 0.0)


def _tc_mlp(x, agg, W1, b1, W2, b2, gamma, beta, block_rows=400):
    n, d = x.shape
    grid = (n // block_rows,)
    row_spec = pl.BlockSpec((block_rows, d), lambda i: (i, 0))
    a0_spec = pl.BlockSpec((1, block_rows, d), lambda i: (0, i, 0))
    a1_spec = pl.BlockSpec((1, block_rows, d), lambda i: (1, i, 0))
    full_spec = pl.BlockSpec((d, d), lambda i: (0, 0))
    vec_spec = pl.BlockSpec((1, d), lambda i: (0, 0))
    return pl.pallas_call(
        _tc_block,
        grid=grid,
        in_specs=[row_spec, a0_spec, a1_spec, full_spec, vec_spec,
                  full_spec, vec_spec, vec_spec, vec_spec],
        out_specs=row_spec,
        out_shape=jax.ShapeDtypeStruct((n, d), jnp.float32),
    )(x, agg, agg, W1, b1.reshape(1, d), W2, b2.reshape(1, d),
      gamma.reshape(1, d), beta.reshape(1, d))


def kernel(x, edge_index, W1, b1, W2, b2, gamma, beta):
    n, d = x.shape
    e = edge_index.shape[1]
    quantum = NW * CH * NIDX
    e_pad = -(-e // quantum) * quantum
    pad = e_pad - e
    src2 = edge_index[0].reshape(NW, e // NW)
    dst2 = edge_index[1].reshape(NW, e // NW)
    if pad:
        ar = jnp.arange(pad, dtype=jnp.int32)
        src2 = jnp.concatenate(
            [src2, (ar % n).reshape(NW, pad // NW)], axis=1)
        dst2 = jnp.concatenate(
            [dst2, (n + ar % DUMMY).reshape(NW, pad // NW)], axis=1)
    epw = e_pad // NW
    nchunk = epw // CH
    src3 = src2.reshape(NW, nchunk, CH)
    dst3 = dst2.reshape(NW, nchunk, CH)
    zeros = jnp.zeros(((n // NS) // 8 * 8, d), dtype=jnp.float32)
    agg = _sc_aggregate(src3, dst3, x, zeros)
    return _tc_mlp(x, agg, W1, b1, W2, b2, gamma, beta)
